# baseline (speedup 1.0000x reference)
import functools

import jax
import jax.numpy as jnp
from jax import lax
from jax.experimental import pallas as pl
from jax.experimental.pallas import tpu as pltpu
from jax.experimental.pallas import tpu_sc as plsc

_B = 512
_D = 64
_H = 128
_MAXN = 512
_TOTAL = _B * (_B - 1) // 2

_TOK_BLK = 256
_N_BLK = _TOTAL // _TOK_BLK

_CHUNK = 128
_NCHUNK = _TOTAL // _CHUNK
_NWORKER = 32


def _mish(v):
    return v * jnp.tanh(jnp.logaddexp(v, 0.0))


def _ln(v, g, b):
    m = jnp.mean(v, axis=-1, keepdims=True)
    var = jnp.var(v, axis=-1, keepdims=True)
    return (v - m) / jnp.sqrt(var + 1e-5) * g + b


def _feat_body(x_ref, wrt_ref, br_ref, v1t_ref, vb1_ref, gv_ref, bv_ref,
               v2t_ref, vb2_ref, mag_ref, mx_ref, vf_ref):
    xb = x_ref[...]
    m = jnp.dot(xb, wrt_ref[...]) + br_ref[0, 0]
    mag_ref[...] = m
    bm = jnp.max(m, axis=0, keepdims=True)
    i = pl.program_id(0)

    @pl.when(i == 0)
    def _():
        mx_ref[...] = bm

    @pl.when(i > 0)
    def _():
        mx_ref[...] = jnp.maximum(mx_ref[...], bm)

    h = jnp.dot(xb, v1t_ref[...]) + vb1_ref[...]
    h = _mish(_ln(h, gv_ref[...], bv_ref[...]))
    vf_ref[...] = jnp.dot(h, v2t_ref[...]) + vb2_ref[...]


def _features(x, W_rank, b_rank, V1, vb1, gv, bv, V2, vb2):
    full = lambda i: (0, 0)
    return pl.pallas_call(
        _feat_body,
        grid=(_N_BLK,),
        in_specs=[
            pl.BlockSpec((_TOK_BLK, _D), lambda i: (i, 0)),
            pl.BlockSpec((_D, 1), full),
            pl.BlockSpec((1, 1), full),
            pl.BlockSpec((_D, 96), full),
            pl.BlockSpec((1, 96), full),
            pl.BlockSpec((1, 96), full),
            pl.BlockSpec((1, 96), full),
            pl.BlockSpec((96, _H), full),
            pl.BlockSpec((1, _H), full),
        ],
        out_specs=[
            pl.BlockSpec((_TOK_BLK, 1), lambda i: (i, 0)),
            pl.BlockSpec((1, 1), full),
            pl.BlockSpec((_TOK_BLK, _H), lambda i: (i, 0)),
        ],
        out_shape=[
            jax.ShapeDtypeStruct((_TOTAL, 1), jnp.float32),
            jax.ShapeDtypeStruct((1, 1), jnp.float32),
            jax.ShapeDtypeStruct((_TOTAL, _H), jnp.float32),
        ],
    )(x, W_rank.T, b_rank.reshape(1, 1), V1.T, vb1.reshape(1, -1),
      gv.reshape(1, -1), bv.reshape(1, -1), V2.T, vb2.reshape(1, -1))


def _kf_body(w1t_ref, b1_ref, g1_ref, be1_ref, w2t_ref, b2_ref, out_ref):
    h = w1t_ref[...] + b1_ref[...]
    h = _mish(_ln(h, g1_ref[...], be1_ref[...]))
    out_ref[...] = jnp.dot(h, w2t_ref[...]) + b2_ref[...]


def _kf_table(W1, b1, g1, be1, W2, b2):
    return pl.pallas_call(
        _kf_body,
        out_shape=jax.ShapeDtypeStruct((_MAXN, _H), jnp.float32),
    )(W1.T, b1.reshape(1, -1), g1.reshape(1, -1), be1.reshape(1, -1),
      W2.T, b2.reshape(1, -1))


def _sc_body(vf_hbm, kf_hbm, perm_hbm, key_hbm, batch_hbm, out_hbm,
             pidx, kidx, bidx, vrows, krows, zstage, zacc, sem1, sem2):
    c = lax.axis_index("c")
    s = lax.axis_index("s")
    wid = s * 2 + c

    def _zrow(r, _):
        for j in range(8):
            zstage[r, pl.ds(j * 16, 16)] = jnp.zeros((16,), jnp.float32)
        return 0

    lax.fori_loop(0, 64, _zrow, 0)

    @pl.when(s == 0)
    def _():
        for k in range(8):
            pltpu.sync_copy(zstage, zacc.at[pl.ds(k * 64, 64)])

    plsc.subcore_barrier()

    nw = (_NCHUNK - wid + _NWORKER - 1) // _NWORKER

    def _chunk(t, _):
        base = (wid + t * _NWORKER) * _CHUNK
        pltpu.sync_copy(perm_hbm.at[pl.ds(base, _CHUNK)], pidx)
        pltpu.sync_copy(key_hbm.at[pl.ds(base, _CHUNK)], kidx)
        pltpu.sync_copy(batch_hbm.at[pl.ds(base, _CHUNK)], bidx)
        cp1 = pltpu.async_copy(vf_hbm.at[pidx], vrows, sem1)
        cp2 = pltpu.async_copy(kf_hbm.at[kidx], krows, sem2)
        cp1.wait()
        cp2.wait()

        def _mul(r, _):
            for j in range(8):
                sl = pl.ds(j * 16, 16)
                vrows[r, sl] = vrows[r, sl] * krows[r, sl]
            return 0

        lax.fori_loop(0, _CHUNK, _mul, 0)
        pltpu.sync_copy(vrows, zacc.at[bidx], add=True)
        return 0

    lax.fori_loop(0, nw, _chunk, 0)
    plsc.subcore_barrier()
    rows = _B // 16
    pltpu.sync_copy(zacc.at[pl.ds(s * rows, rows)],
                    out_hbm.at[c, pl.ds(s * rows, rows)])


def _sc_combine(vf, kf, perm, keys, batch):
    mesh = plsc.VectorSubcoreMesh(core_axis_name="c", subcore_axis_name="s")
    f = functools.partial(
        pl.kernel,
        out_type=jax.ShapeDtypeStruct((2, _B, _H), jnp.float32),
        mesh=mesh,
        scratch_types=[
            pltpu.VMEM((_CHUNK,), jnp.int32),
            pltpu.VMEM((_CHUNK,), jnp.int32),
            pltpu.VMEM((_CHUNK,), jnp.int32),
            pltpu.VMEM((_CHUNK, _H), jnp.float32),
            pltpu.VMEM((_CHUNK, _H), jnp.float32),
            pltpu.VMEM((64, _H), jnp.float32),
            pltpu.VMEM_SHARED((_B, _H), jnp.float32),
            pltpu.SemaphoreType.DMA,
            pltpu.SemaphoreType.DMA,
        ],
    )(_sc_body)
    return f(vf, kf, perm, keys, batch)


_NSORT = 131072
_SROWS = 1024
_SCOLS = 128


def _sort_stages():
    st = []
    k = 2
    while k <= _NSORT:
        jj = k // 2
        while jj >= 1:
            msel = max(jj // _SROWS, 1).bit_length() - 1
            st.append((k, jj, msel))
            jj //= 2
        k *= 2
    return st

_STAGES = _sort_stages()
_NSTAGE = len(_STAGES)


def _sort_body(keyin_ref, out_ref, xk, xi):
    p_idx = pl.program_id(0)
    s_idx = pl.program_id(1)
    iota_r = lax.broadcasted_iota(jnp.int32, (_SROWS, _SCOLS), 0)
    iota_c = lax.broadcasted_iota(jnp.int32, (_SROWS, _SCOLS), 1)

    @pl.when((p_idx == 0) & (s_idx == 0))
    def _():
        xk[...] = keyin_ref[...]
        xi[...] = (iota_r * _SCOLS + iota_c).astype(jnp.float32)

    @pl.when(s_idx <= p_idx)
    def _():
        k_s = jnp.int32(1) << (p_idx + 1)
        jj_s = jnp.int32(1) << (p_idx - s_idx)
        m_row = jnp.where(jj_s < _SROWS, jj_s, 0)
        m_col = jnp.where(jj_s < _SROWS, 0, jj_s // _SROWS)

        ra = lax.broadcasted_iota(jnp.int32, (_SROWS, _SROWS), 0)
        rb = lax.broadcasted_iota(jnp.int32, (_SROWS, _SROWS), 1)
        prow = ((ra ^ rb) == m_row).astype(jnp.float32)
        ca = lax.broadcasted_iota(jnp.int32, (_SCOLS, _SCOLS), 0)
        cb = lax.broadcasted_iota(jnp.int32, (_SCOLS, _SCOLS), 1)
        qcol = ((ca ^ cb) == m_col).astype(jnp.float32)

        hp = jax.lax.Precision.HIGHEST
        a_k = xk[...]
        a_i = xi[...]
        b_k = jnp.dot(jnp.dot(prow, a_k, precision=hp), qcol, precision=hp)
        b_i = jnp.dot(jnp.dot(prow, a_i, precision=hp), qcol, precision=hp)

        ilog = iota_c * _SROWS + iota_r
        asc = (ilog & k_s) == 0
        low = (ilog & jj_s) == 0
        cmp = (a_k > b_k) | ((a_k == b_k) & (a_i > b_i))
        take = jnp.logical_not(jnp.logical_xor(
            jnp.logical_not(jnp.logical_xor(cmp, asc)), low))
        xk[...] = jnp.where(take, b_k, a_k)
        xi[...] = jnp.where(take, b_i, a_i)

    @pl.when((p_idx == 16) & (s_idx == 16))
    def _():
        out_ref[...] = xi[...]


def _bitonic_argsort(new_mag):
    keyin = jnp.concatenate(
        [new_mag, jnp.full((_NSORT - _TOTAL,), 1e30, jnp.float32)]
    ).reshape(_SROWS, _SCOLS)
    full = lambda p, s: (0, 0)
    out = pl.pallas_call(
        _sort_body,
        grid=(17, 17),
        in_specs=[pl.BlockSpec((_SROWS, _SCOLS), full)],
        out_specs=pl.BlockSpec((_SROWS, _SCOLS), full),
        out_shape=jax.ShapeDtypeStruct((_SROWS, _SCOLS), jnp.float32),
        scratch_shapes=[
            pltpu.VMEM((_SROWS, _SCOLS), jnp.float32),
            pltpu.VMEM((_SROWS, _SCOLS), jnp.float32),
        ],
    )(keyin)
    return out.T.reshape(-1)[:_TOTAL].astype(jnp.int32)


def _epi_body(z0_ref, z1_ref, nf_ref, wc_ref, bc_ref, out_ref):
    out_ref[...] = (z0_ref[...] + z1_ref[...]
                    + nf_ref[...] * wc_ref[...] + bc_ref[...])


def _epilogue(zp, n, W_card, b_card):
    return pl.pallas_call(
        _epi_body,
        out_shape=jax.ShapeDtypeStruct((_B, _H), jnp.float32),
    )(zp[0], zp[1], n.astype(jnp.float32).reshape(_B, 1),
      W_card.reshape(1, _H), b_card.reshape(1, _H))


def kernel(x, n, W_rank, b_rank, W1, b1, g1, be1, W2, b2,
           V1, vb1, gv, bv, V2, vb2, W_card, b_card):
    total = x.shape[0]
    nb = n.shape[0]

    mag, mx, vf = _features(x, W_rank, b_rank, V1, vb1, gv, bv, V2, vb2)
    kf = _kf_table(W1, b1, g1, be1, W2, b2)

    batch = jnp.repeat(jnp.arange(nb), n, total_repeat_length=total)
    csum = jnp.cumsum(n)
    offsets = csum - n
    keys = (jnp.arange(total)
            - jnp.repeat(offsets, n, total_repeat_length=total)).astype(jnp.int32)

    max_mag = mx[0, 0] + 0.0001
    new_mag = mag[:, 0] + batch.astype(x.dtype) * max_mag
    perm = _bitonic_argsort(new_mag)

    zp = _sc_combine(vf, kf, perm, keys, batch.astype(jnp.int32))
    return _epilogue(zp, n, W_card, b_card)


# speedup vs baseline: 1.9153x; 1.9153x over previous
import functools

import jax
import jax.numpy as jnp
from jax import lax
from jax.experimental import pallas as pl
from jax.experimental.pallas import tpu as pltpu
from jax.experimental.pallas import tpu_sc as plsc

_B = 512
_D = 64
_H = 128
_MAXN = 512
_TOTAL = _B * (_B - 1) // 2

_TOK_BLK = 256
_N_BLK = _TOTAL // _TOK_BLK

_CHUNK = 128
_NCHUNK = _TOTAL // _CHUNK
_NWORKER = 32


def _mish(v):
    return v * jnp.tanh(jnp.logaddexp(v, 0.0))


def _ln(v, g, b):
    m = jnp.mean(v, axis=-1, keepdims=True)
    var = jnp.var(v, axis=-1, keepdims=True)
    return (v - m) / jnp.sqrt(var + 1e-5) * g + b


def _feat_body(x_ref, wrt_ref, br_ref, v1t_ref, vb1_ref, gv_ref, bv_ref,
               v2t_ref, vb2_ref, mag_ref, mx_ref, vf_ref):
    xb = x_ref[...]
    m = jnp.dot(xb, wrt_ref[...]) + br_ref[0, 0]
    mag_ref[...] = m
    bm = jnp.max(m, axis=0, keepdims=True)
    i = pl.program_id(0)

    @pl.when(i == 0)
    def _():
        mx_ref[...] = bm

    @pl.when(i > 0)
    def _():
        mx_ref[...] = jnp.maximum(mx_ref[...], bm)

    h = jnp.dot(xb, v1t_ref[...]) + vb1_ref[...]
    h = _mish(_ln(h, gv_ref[...], bv_ref[...]))
    vf_ref[...] = jnp.dot(h, v2t_ref[...]) + vb2_ref[...]


def _features(x, W_rank, b_rank, V1, vb1, gv, bv, V2, vb2):
    full = lambda i: (0, 0)
    return pl.pallas_call(
        _feat_body,
        grid=(_N_BLK,),
        in_specs=[
            pl.BlockSpec((_TOK_BLK, _D), lambda i: (i, 0)),
            pl.BlockSpec((_D, 1), full),
            pl.BlockSpec((1, 1), full),
            pl.BlockSpec((_D, 96), full),
            pl.BlockSpec((1, 96), full),
            pl.BlockSpec((1, 96), full),
            pl.BlockSpec((1, 96), full),
            pl.BlockSpec((96, _H), full),
            pl.BlockSpec((1, _H), full),
        ],
        out_specs=[
            pl.BlockSpec((_TOK_BLK, 1), lambda i: (i, 0)),
            pl.BlockSpec((1, 1), full),
            pl.BlockSpec((_TOK_BLK, _H), lambda i: (i, 0)),
        ],
        out_shape=[
            jax.ShapeDtypeStruct((_TOTAL, 1), jnp.float32),
            jax.ShapeDtypeStruct((1, 1), jnp.float32),
            jax.ShapeDtypeStruct((_TOTAL, _H), jnp.float32),
        ],
    )(x, W_rank.T, b_rank.reshape(1, 1), V1.T, vb1.reshape(1, -1),
      gv.reshape(1, -1), bv.reshape(1, -1), V2.T, vb2.reshape(1, -1))


def _kf_body(w1t_ref, b1_ref, g1_ref, be1_ref, w2t_ref, b2_ref, out_ref):
    h = w1t_ref[...] + b1_ref[...]
    h = _mish(_ln(h, g1_ref[...], be1_ref[...]))
    out_ref[...] = jnp.dot(h, w2t_ref[...]) + b2_ref[...]


def _kf_table(W1, b1, g1, be1, W2, b2):
    return pl.pallas_call(
        _kf_body,
        out_shape=jax.ShapeDtypeStruct((_MAXN, _H), jnp.float32),
    )(W1.T, b1.reshape(1, -1), g1.reshape(1, -1), be1.reshape(1, -1),
      W2.T, b2.reshape(1, -1))


def _sc_body(vf_hbm, kf_hbm, perm_hbm, key_hbm, batch_hbm, out_hbm,
             pidx, kidx, bidx, vrows, krows, zstage, zacc, sem1, sem2):
    c = lax.axis_index("c")
    s = lax.axis_index("s")
    wid = s * 2 + c

    def _zrow(r, _):
        for j in range(8):
            zstage[r, pl.ds(j * 16, 16)] = jnp.zeros((16,), jnp.float32)
        return 0

    lax.fori_loop(0, 64, _zrow, 0)

    @pl.when(s == 0)
    def _():
        for k in range(8):
            pltpu.sync_copy(zstage, zacc.at[pl.ds(k * 64, 64)])

    plsc.subcore_barrier()

    nw = (_NCHUNK - wid + _NWORKER - 1) // _NWORKER

    def _chunk(t, _):
        base = (wid + t * _NWORKER) * _CHUNK
        pltpu.sync_copy(perm_hbm.at[pl.ds(base, _CHUNK)], pidx)
        pltpu.sync_copy(key_hbm.at[pl.ds(base, _CHUNK)], kidx)
        pltpu.sync_copy(batch_hbm.at[pl.ds(base, _CHUNK)], bidx)
        cp1 = pltpu.async_copy(vf_hbm.at[pidx], vrows, sem1)
        cp2 = pltpu.async_copy(kf_hbm.at[kidx], krows, sem2)
        cp1.wait()
        cp2.wait()

        def _mul(r, _):
            for j in range(8):
                sl = pl.ds(j * 16, 16)
                vrows[r, sl] = vrows[r, sl] * krows[r, sl]
            return 0

        lax.fori_loop(0, _CHUNK, _mul, 0)
        pltpu.sync_copy(vrows, zacc.at[bidx], add=True)
        return 0

    lax.fori_loop(0, nw, _chunk, 0)
    plsc.subcore_barrier()
    rows = _B // 16
    pltpu.sync_copy(zacc.at[pl.ds(s * rows, rows)],
                    out_hbm.at[c, pl.ds(s * rows, rows)])


def _sc_combine(vf, kf, perm, keys, batch):
    mesh = plsc.VectorSubcoreMesh(core_axis_name="c", subcore_axis_name="s")
    f = functools.partial(
        pl.kernel,
        out_type=jax.ShapeDtypeStruct((2, _B, _H), jnp.float32),
        mesh=mesh,
        scratch_types=[
            pltpu.VMEM((_CHUNK,), jnp.int32),
            pltpu.VMEM((_CHUNK,), jnp.int32),
            pltpu.VMEM((_CHUNK,), jnp.int32),
            pltpu.VMEM((_CHUNK, _H), jnp.float32),
            pltpu.VMEM((_CHUNK, _H), jnp.float32),
            pltpu.VMEM((64, _H), jnp.float32),
            pltpu.VMEM_SHARED((_B, _H), jnp.float32),
            pltpu.SemaphoreType.DMA,
            pltpu.SemaphoreType.DMA,
        ],
    )(_sc_body)
    return f(vf, kf, perm, keys, batch)


_NSORT = 131072
_SROWS = 1024
_SCOLS = 128


def _sort_stages():
    st = []
    k = 2
    while k <= _NSORT:
        jj = k // 2
        while jj >= 1:
            msel = max(jj // _SROWS, 1).bit_length() - 1
            st.append((k, jj, msel))
            jj //= 2
        k *= 2
    return st

_STAGES = _sort_stages()
_NSTAGE = len(_STAGES)


def _sort_body(karr_ref, jjarr_ref, msel_ref, keyin_ref, p_ref, out_ref,
               xk, xi, yk, yi):
    t = pl.program_id(0)
    iota_r = lax.broadcasted_iota(jnp.int32, (_SROWS, _SCOLS), 0)
    iota_c = lax.broadcasted_iota(jnp.int32, (_SROWS, _SCOLS), 1)

    @pl.when(t == 0)
    def _():
        xk[...] = keyin_ref[...]
        xi[...] = (iota_r * _SCOLS + iota_c).astype(jnp.float32)

    k_s = karr_ref[t]
    jj_s = jjarr_ref[t]
    msel = msel_ref[t]

    for m in (1, 2, 4, 8, 16, 32, 64, 128, 256, 512):
        @pl.when(jj_s == m)
        def _(m=m):
            q = _SROWS // (2 * m)
            zk = xk[...].reshape(q, 2 * m, _SCOLS)
            zi = xi[...].reshape(q, 2 * m, _SCOLS)
            yk[...] = jnp.concatenate([zk[:, m:], zk[:, :m]], 1).reshape(
                _SROWS, _SCOLS)
            yi[...] = jnp.concatenate([zi[:, m:], zi[:, :m]], 1).reshape(
                _SROWS, _SCOLS)

    @pl.when(jj_s >= _SROWS)
    def _():
        P = p_ref[msel]
        yk[...] = jnp.dot(xk[...], P, preferred_element_type=jnp.float32,
                          precision=jax.lax.Precision.HIGHEST)
        yi[...] = jnp.dot(xi[...], P, preferred_element_type=jnp.float32,
                          precision=jax.lax.Precision.HIGHEST)

    ilog = iota_c * _SROWS + iota_r
    asc = (ilog & k_s) == 0
    low = (ilog & jj_s) == 0
    a_k = xk[...]
    a_i = xi[...]
    b_k = yk[...]
    b_i = yi[...]
    cmp = (a_k > b_k) | ((a_k == b_k) & (a_i > b_i))
    take = jnp.logical_not(jnp.logical_xor(
        jnp.logical_not(jnp.logical_xor(cmp, asc)), low))
    xk[...] = jnp.where(take, b_k, a_k)
    xi[...] = jnp.where(take, b_i, a_i)

    @pl.when(t == _NSTAGE - 1)
    def _():
        out_ref[...] = xi[...]


def _bitonic_argsort(new_mag):
    import numpy as np
    st = np.array(_STAGES, dtype=np.int32)
    karr = jnp.asarray(st[:, 0])
    jjarr = jnp.asarray(st[:, 1])
    msarr = jnp.asarray(st[:, 2])
    cols = np.arange(_SCOLS)
    pmats = jnp.asarray(np.stack(
        [np.eye(_SCOLS, dtype=np.float32)[cols ^ (1 << b)] for b in range(7)]))
    keyin = jnp.concatenate(
        [new_mag, jnp.full((_NSORT - _TOTAL,), 1e30, jnp.float32)]
    ).reshape(_SROWS, _SCOLS)
    full = lambda t, *_: (0, 0)
    out = pl.pallas_call(
        _sort_body,
        grid_spec=pltpu.PrefetchScalarGridSpec(
            num_scalar_prefetch=3,
            grid=(_NSTAGE,),
            in_specs=[
                pl.BlockSpec((_SROWS, _SCOLS), full),
                pl.BlockSpec((7, _SCOLS, _SCOLS), lambda t, *_: (0, 0, 0)),
            ],
            out_specs=pl.BlockSpec((_SROWS, _SCOLS), full),
            scratch_shapes=[
                pltpu.VMEM((_SROWS, _SCOLS), jnp.float32),
                pltpu.VMEM((_SROWS, _SCOLS), jnp.float32),
                pltpu.VMEM((_SROWS, _SCOLS), jnp.float32),
                pltpu.VMEM((_SROWS, _SCOLS), jnp.float32),
            ],
        ),
        out_shape=jax.ShapeDtypeStruct((_SROWS, _SCOLS), jnp.float32),
    )(karr, jjarr, msarr, keyin, pmats)
    return out.T.reshape(-1)[:_TOTAL].astype(jnp.int32)


def _epi_body(z0_ref, z1_ref, nf_ref, wc_ref, bc_ref, out_ref):
    out_ref[...] = (z0_ref[...] + z1_ref[...]
                    + nf_ref[...] * wc_ref[...] + bc_ref[...])


def _epilogue(zp, n, W_card, b_card):
    return pl.pallas_call(
        _epi_body,
        out_shape=jax.ShapeDtypeStruct((_B, _H), jnp.float32),
    )(zp[0], zp[1], n.astype(jnp.float32).reshape(_B, 1),
      W_card.reshape(1, _H), b_card.reshape(1, _H))


def kernel(x, n, W_rank, b_rank, W1, b1, g1, be1, W2, b2,
           V1, vb1, gv, bv, V2, vb2, W_card, b_card):
    total = x.shape[0]
    nb = n.shape[0]

    mag, mx, vf = _features(x, W_rank, b_rank, V1, vb1, gv, bv, V2, vb2)
    kf = _kf_table(W1, b1, g1, be1, W2, b2)

    batch = jnp.repeat(jnp.arange(nb), n, total_repeat_length=total)
    csum = jnp.cumsum(n)
    offsets = csum - n
    keys = (jnp.arange(total)
            - jnp.repeat(offsets, n, total_repeat_length=total)).astype(jnp.int32)

    max_mag = mx[0, 0] + 0.0001
    new_mag = mag[:, 0] + batch.astype(x.dtype) * max_mag
    perm = _bitonic_argsort(new_mag)

    zp = _sc_combine(vf, kf, perm, keys, batch.astype(jnp.int32))
    return _epilogue(zp, n, W_card, b_card)
